# baseline (speedup 1.0000x reference)
import functools

import jax
import jax.numpy as jnp
from jax import lax
from jax.experimental import pallas as pl
from jax.experimental.pallas import tpu as pltpu
from jax.experimental.pallas import tpu_sc as plsc

NC = 2
NS = 16
NW = NC * NS
L = 16
CH = 50


def _zero_rows(buf, nrows, ncols):
  z = jnp.zeros((L,), jnp.float32)

  def body(i, _):
    for jj in range(ncols // L):
      buf[i, pl.ds(jj * L, L)] = z
    return 0

  lax.fori_loop(0, nrows, body, 0)


def _fill_ones(buf, nrows, ncols):
  o = jnp.ones((L,), jnp.float32)

  def body(i, _):
    for jj in range(ncols // L):
      buf[i, pl.ds(jj * L, L)] = o
    return 0

  lax.fori_loop(0, nrows, body, 0)


def _zero_slab(zsrc, zrows, acc, row0, npt):
  off = 0
  while off < npt:
    step = min(zrows, npt - off)
    pltpu.sync_copy(zsrc.at[pl.ds(0, step)],
                    acc.at[pl.ds(row0 + off, step)])
    off += step


def _make_deg_kernel(n, e):
  ring = 8
  assert e % (CH * NW) == 0 and n % (8 * NS) == 0
  nch = e // CH
  ncw = nch // NW
  assert ncw >= ring
  npt = n // NS
  zr = min(npt, CH)
  mesh = plsc.VectorSubcoreMesh(core_axis_name="c", subcore_axis_name="s")

  @functools.partial(
      pl.kernel,
      out_type=jax.ShapeDtypeStruct((NC, n, L), jnp.float32),
      mesh=mesh,
      compiler_params=pltpu.CompilerParams(use_tc_tiling_on_sc=False),
      scratch_types=[
          pltpu.VMEM_SHARED((n, L), jnp.float32),
          pltpu.VMEM((CH, L), jnp.float32),
          pltpu.VMEM((ncw, CH), jnp.int32),
          pltpu.SemaphoreType.DMA((ring,)),
      ],
  )
  def deg_kernel(dst2_hbm, degp_hbm, acc, buf, didx_all, ssem):
    c = lax.axis_index("c")
    s = lax.axis_index("s")
    w = c * NS + s
    row0 = s * npt

    pltpu.sync_copy(dst2_hbm.at[pl.ds(w * ncw, ncw)], didx_all)

    _zero_rows(buf, zr, L)
    _zero_slab(buf, zr, acc, row0, npt)
    plsc.subcore_barrier()

    _fill_ones(buf, CH, L)
    for jj in range(ring):
      pltpu.async_copy(buf, acc.at[didx_all.at[jj]], ssem.at[jj],
                       add=True)

    def body(j, _):
      rb = j % ring
      pltpu.make_async_copy(buf, acc.at[didx_all.at[j - ring]],
                            ssem.at[rb]).wait()
      pltpu.async_copy(buf, acc.at[didx_all.at[j]], ssem.at[rb], add=True)
      return 0

    lax.fori_loop(ring, ncw, body, 0)
    for k in range(ring):
      pltpu.make_async_copy(buf, acc.at[didx_all.at[ncw - ring + k]],
                            ssem.at[(ncw - ring + k) % ring]).wait()
    plsc.subcore_barrier()
    pltpu.sync_copy(acc.at[pl.ds(row0, npt)],
                    degp_hbm.at[c, pl.ds(row0, npt)])

  return deg_kernel


def _make_agg_kernel(n, e, dh):
  ring, lead = 8, 4
  assert e % (CH * NS) == 0 and n % (8 * NS) == 0 and dh % L == 0
  nch = e // CH
  ncw = nch // NS
  assert ncw % ring == 0 and ncw >= 2 * ring
  npt = n // NS
  zr = min(npt, CH)
  mesh = plsc.VectorSubcoreMesh(core_axis_name="c", subcore_axis_name="s")

  @functools.partial(
      pl.kernel,
      out_type=jax.ShapeDtypeStruct((NC, n, dh), jnp.float32),
      mesh=mesh,
      compiler_params=pltpu.CompilerParams(use_tc_tiling_on_sc=False),
      scratch_types=[
          pltpu.VMEM_SHARED((n, dh), jnp.float32),
          pltpu.VMEM((ring, CH, dh), jnp.float32),
          pltpu.VMEM((ncw, CH), jnp.int32),
          pltpu.VMEM((ncw, CH), jnp.int32),
          pltpu.SemaphoreType.DMA((ring,)),
          pltpu.SemaphoreType.DMA((ring,)),
      ],
  )
  def agg_kernel(y_hbm, srcs_hbm, dst2_hbm, aggp_hbm,
                 acc, rows_v, sidx_all, didx_all, gsem, ssem):
    c = lax.axis_index("c")
    s = lax.axis_index("s")
    row0 = s * npt

    pltpu.sync_copy(srcs_hbm.at[c, pl.ds(s * ncw, ncw)], sidx_all)
    pltpu.sync_copy(dst2_hbm.at[pl.ds(s * ncw, ncw)], didx_all)

    zslot = rows_v.at[0]
    _zero_rows(zslot, zr, dh)
    _zero_slab(zslot, zr, acc, row0, npt)
    plsc.subcore_barrier()

    def fire_gather(j, slot):
      pltpu.async_copy(y_hbm.at[sidx_all.at[j]], rows_v.at[slot],
                       gsem.at[slot])

    def wait_gather(j, slot):
      pltpu.make_async_copy(y_hbm.at[sidx_all.at[j]], rows_v.at[slot],
                            gsem.at[slot]).wait()

    def fire_scatter(j, slot):
      pltpu.async_copy(rows_v.at[slot], acc.at[didx_all.at[j]],
                       ssem.at[slot], add=True)

    def wait_scatter(j, slot):
      pltpu.make_async_copy(rows_v.at[slot], acc.at[didx_all.at[j]],
                            ssem.at[slot]).wait()

    for j in range(lead):
      fire_gather(j, j)
    for j in range(ring - lead):
      wait_gather(j, j % ring)
      fire_scatter(j, j % ring)
      fire_gather(j + lead, (j + lead) % ring)

    def body(j, _):
      rb = j % ring
      wait_gather(j, rb)
      fire_scatter(j, rb)
      k = j + lead
      rk = k % ring
      wait_scatter(k - ring, rk)
      fire_gather(k, rk)
      return 0

    lax.fori_loop(ring - lead, ncw - lead, body, 0)

    def tail(j, _):
      rb = j % ring
      wait_gather(j, rb)
      fire_scatter(j, rb)
      return 0

    lax.fori_loop(ncw - lead, ncw, tail, 0)
    for k in range(ring):
      j = ncw - ring + k
      wait_scatter(j, j % ring)
    plsc.subcore_barrier()
    pltpu.sync_copy(acc.at[pl.ds(row0, npt)],
                    aggp_hbm.at[c, pl.ds(row0, npt)])

  return agg_kernel


def _tc_layer1(degp0_ref, degp1_ref, x_ref, w1_ref, dinv_ref, y1_ref):
  deg = degp0_ref[...] + degp1_ref[...] + 1.0
  dinv = lax.rsqrt(deg)
  dinv_ref[...] = dinv
  xw = jnp.dot(x_ref[...], w1_ref[0], preferred_element_type=jnp.float32)
  y1_ref[0] = xw * dinv[:, 0:1]


def _tc_layer2(a10_ref, a11_ref, y10_ref, y11_ref, dinv_ref, w2_ref,
               y2_ref):
  dv = dinv_ref[...][:, 0:1]
  hfull = jnp.concatenate(
      [a10_ref[...] + y10_ref[...], a11_ref[...] + y11_ref[...]], axis=1)
  hact = jnp.tanh(hfull * dv)
  y2_ref[0] = jnp.dot(hact, w2_ref[0],
                      preferred_element_type=jnp.float32) * dv


def _tc_final(a20_ref, a21_ref, y20_ref, y21_ref, dinv_ref, out_ref):
  dv = dinv_ref[...][:, 0:1]
  out_ref[...] = jnp.concatenate(
      [a20_ref[...] + y20_ref[...], a21_ref[...] + y21_ref[...]],
      axis=1) * dv


def kernel(x, edge_index, W1, W2):
  n, f_in = x.shape
  e = edge_index.shape[1]
  h = W1.shape[1]
  cdim = W2.shape[1]
  cpad = 128
  dh1 = h // 2
  dh2 = cpad // 2
  assert e % CH == 0
  nch = e // CH
  src2 = edge_index[0].reshape(nch, CH)
  dst2 = edge_index[1].reshape(nch, CH)
  srcs = jnp.stack([src2, src2 + n])
  W2p = jnp.zeros((h, cpad), jnp.float32).at[:, :cdim].set(W2)
  W1s = jnp.stack([W1[:, :dh1], W1[:, dh1:]])
  W2s = jnp.stack([W2p[:, :dh2], W2p[:, dh2:]])
  np_pad = -(-n // (8 * NS)) * (8 * NS)

  blk = 2000
  assert n % blk == 0
  nb = n // blk
  row = lambda width: pl.BlockSpec((blk, width), lambda c, i: (i, 0))
  plane = lambda width: pl.BlockSpec((1, blk, width), lambda c, i: (c, i, 0))
  wcol = lambda r, width: pl.BlockSpec((1, r, width), lambda c, i: (c, 0, 0))

  degp = _make_deg_kernel(np_pad, e)(dst2)

  dinv, y1 = pl.pallas_call(
      _tc_layer1,
      grid=(2, nb),
      in_specs=[row(L), row(L), row(f_in), wcol(f_in, dh1)],
      out_specs=[row(L), plane(dh1)],
      out_shape=[
          jax.ShapeDtypeStruct((n, L), jnp.float32),
          jax.ShapeDtypeStruct((2, n, dh1), jnp.float32),
      ],
  )(degp[0], degp[1], x, W1s)

  aggp1 = _make_agg_kernel(np_pad, e, dh1)(y1.reshape(2 * n, dh1),
                                           srcs, dst2)

  y2 = pl.pallas_call(
      _tc_layer2,
      grid=(2, nb),
      in_specs=[row(dh1), row(dh1), row(dh1), row(dh1), row(L),
                wcol(h, dh2)],
      out_specs=plane(dh2),
      out_shape=jax.ShapeDtypeStruct((2, n, dh2), jnp.float32),
  )(aggp1[0], aggp1[1], y1[0], y1[1], dinv, W2s)

  aggp2 = _make_agg_kernel(np_pad, e, dh2)(y2.reshape(2 * n, dh2),
                                           srcs, dst2)

  rowf = lambda width: pl.BlockSpec((blk, width), lambda i: (i, 0))
  out = pl.pallas_call(
      _tc_final,
      grid=(nb,),
      in_specs=[rowf(dh2), rowf(dh2), rowf(dh2), rowf(dh2), rowf(L)],
      out_specs=rowf(cpad),
      out_shape=jax.ShapeDtypeStruct((n, cpad), jnp.float32),
  )(aggp2[0], aggp2[1], y2[0], y2[1], dinv)

  return out[:, :cdim]


# speedup vs baseline: 1.1087x; 1.1087x over previous
import functools

import jax
import jax.numpy as jnp
from jax import lax
from jax.experimental import pallas as pl
from jax.experimental.pallas import tpu as pltpu
from jax.experimental.pallas import tpu_sc as plsc

NC = 2
NS = 16
NW = NC * NS
L = 16
CH = 50


def _zero_rows(buf, nrows, ncols):
  z = jnp.zeros((L,), jnp.float32)

  def body(i, _):
    for jj in range(ncols // L):
      buf[i, pl.ds(jj * L, L)] = z
    return 0

  lax.fori_loop(0, nrows, body, 0)


def _fill_ones(buf, nrows, ncols):
  o = jnp.ones((L,), jnp.float32)

  def body(i, _):
    for jj in range(ncols // L):
      buf[i, pl.ds(jj * L, L)] = o
    return 0

  lax.fori_loop(0, nrows, body, 0)


def _zero_slab(zsrc, zrows, acc, row0, npt):
  off = 0
  while off < npt:
    step = min(zrows, npt - off)
    pltpu.sync_copy(zsrc.at[pl.ds(0, step)],
                    acc.at[pl.ds(row0 + off, step)])
    off += step


def _make_deg_kernel(n, e):
  ring = 8
  assert e % (CH * NW) == 0 and n % (8 * NS) == 0
  nch = e // CH
  ncw = nch // NW
  assert ncw >= ring
  npt = n // NS
  zr = min(npt, CH)
  mesh = plsc.VectorSubcoreMesh(core_axis_name="c", subcore_axis_name="s")

  @functools.partial(
      pl.kernel,
      out_type=jax.ShapeDtypeStruct((NC, n, L), jnp.float32),
      mesh=mesh,
      compiler_params=pltpu.CompilerParams(use_tc_tiling_on_sc=False),
      scratch_types=[
          pltpu.VMEM_SHARED((n, L), jnp.float32),
          pltpu.VMEM((CH, L), jnp.float32),
          pltpu.VMEM((ncw, CH), jnp.int32),
          pltpu.SemaphoreType.DMA((ring,)),
      ],
  )
  def deg_kernel(dst2_hbm, degp_hbm, acc, buf, didx_all, ssem):
    c = lax.axis_index("c")
    s = lax.axis_index("s")
    w = c * NS + s
    row0 = s * npt

    pltpu.sync_copy(dst2_hbm.at[pl.ds(w * ncw, ncw)], didx_all)

    _zero_rows(buf, zr, L)
    _zero_slab(buf, zr, acc, row0, npt)
    plsc.subcore_barrier()

    _fill_ones(buf, CH, L)
    for jj in range(ring):
      pltpu.async_copy(buf, acc.at[didx_all.at[jj]], ssem.at[jj],
                       add=True)

    def body(j, _):
      rb = j % ring
      pltpu.make_async_copy(buf, acc.at[didx_all.at[j - ring]],
                            ssem.at[rb]).wait()
      pltpu.async_copy(buf, acc.at[didx_all.at[j]], ssem.at[rb], add=True)
      return 0

    lax.fori_loop(ring, ncw, body, 0)
    for k in range(ring):
      pltpu.make_async_copy(buf, acc.at[didx_all.at[ncw - ring + k]],
                            ssem.at[(ncw - ring + k) % ring]).wait()
    plsc.subcore_barrier()
    pltpu.sync_copy(acc.at[pl.ds(row0, npt)],
                    degp_hbm.at[c, pl.ds(row0, npt)])

  return deg_kernel


def _make_agg_kernel(n, e, dh, feat_split):
  ring, lead = 8, 4
  assert e % (CH * NW) == 0 and n % (8 * NS) == 0 and dh % L == 0
  nch = e // CH
  ncw = nch // NS if feat_split else nch // NW
  assert ncw % ring == 0 and ncw >= 2 * ring
  npt = n // NS
  zr = min(npt, CH)
  mesh = plsc.VectorSubcoreMesh(core_axis_name="c", subcore_axis_name="s")

  @functools.partial(
      pl.kernel,
      out_type=jax.ShapeDtypeStruct((NC, n, dh), jnp.float32),
      mesh=mesh,
      compiler_params=pltpu.CompilerParams(use_tc_tiling_on_sc=False),
      scratch_types=[
          pltpu.VMEM_SHARED((n, dh), jnp.float32),
          pltpu.VMEM((ring, CH, dh), jnp.float32),
          pltpu.VMEM((ncw, CH), jnp.int32),
          pltpu.VMEM((ncw, CH), jnp.int32),
          pltpu.SemaphoreType.DMA((ring,)),
          pltpu.SemaphoreType.DMA((ring,)),
      ],
  )
  def agg_kernel(y_hbm, srcs_hbm, dst2_hbm, aggp_hbm,
                 acc, rows_v, sidx_all, didx_all, gsem, ssem):
    c = lax.axis_index("c")
    s = lax.axis_index("s")
    row0 = s * npt

    if feat_split:
      pltpu.sync_copy(srcs_hbm.at[c, pl.ds(s * ncw, ncw)], sidx_all)
      pltpu.sync_copy(dst2_hbm.at[pl.ds(s * ncw, ncw)], didx_all)
    else:
      w = c * NS + s
      pltpu.sync_copy(srcs_hbm.at[pl.ds(w * ncw, ncw)], sidx_all)
      pltpu.sync_copy(dst2_hbm.at[pl.ds(w * ncw, ncw)], didx_all)

    zslot = rows_v.at[0]
    _zero_rows(zslot, zr, dh)
    _zero_slab(zslot, zr, acc, row0, npt)
    plsc.subcore_barrier()

    def fire_gather(j, slot):
      pltpu.async_copy(y_hbm.at[sidx_all.at[j]], rows_v.at[slot],
                       gsem.at[slot])

    def wait_gather(j, slot):
      pltpu.make_async_copy(y_hbm.at[sidx_all.at[j]], rows_v.at[slot],
                            gsem.at[slot]).wait()

    def fire_scatter(j, slot):
      pltpu.async_copy(rows_v.at[slot], acc.at[didx_all.at[j]],
                       ssem.at[slot], add=True)

    def wait_scatter(j, slot):
      pltpu.make_async_copy(rows_v.at[slot], acc.at[didx_all.at[j]],
                            ssem.at[slot]).wait()

    for j in range(lead):
      fire_gather(j, j)
    for j in range(ring - lead):
      wait_gather(j, j % ring)
      fire_scatter(j, j % ring)
      fire_gather(j + lead, (j + lead) % ring)

    def body(j, _):
      rb = j % ring
      wait_gather(j, rb)
      fire_scatter(j, rb)
      k = j + lead
      rk = k % ring
      wait_scatter(k - ring, rk)
      fire_gather(k, rk)
      return 0

    lax.fori_loop(ring - lead, ncw - lead, body, 0)

    def tail(j, _):
      rb = j % ring
      wait_gather(j, rb)
      fire_scatter(j, rb)
      return 0

    lax.fori_loop(ncw - lead, ncw, tail, 0)
    for k in range(ring):
      j = ncw - ring + k
      wait_scatter(j, j % ring)
    plsc.subcore_barrier()
    pltpu.sync_copy(acc.at[pl.ds(row0, npt)],
                    aggp_hbm.at[c, pl.ds(row0, npt)])

  return agg_kernel


def _tc_layer1(degp0_ref, degp1_ref, x_ref, w1_ref, dinv_ref, y1_ref):
  deg = degp0_ref[...] + degp1_ref[...] + 1.0
  dinv = lax.rsqrt(deg)
  dinv_ref[...] = dinv
  xw = jnp.dot(x_ref[...], w1_ref[...], preferred_element_type=jnp.float32)
  y1_ref[...] = xw * dinv[:, 0:1]


def _tc_layer2(a10_ref, a11_ref, y1_ref, dinv_ref, w2_ref, y2_ref):
  dv = dinv_ref[...][:, 0:1]
  hfull = a10_ref[...] + a11_ref[...] + y1_ref[...]
  hact = jnp.tanh(hfull * dv)
  y2_ref[0] = jnp.dot(hact, w2_ref[0],
                      preferred_element_type=jnp.float32) * dv


def _tc_final(a20_ref, a21_ref, y20_ref, y21_ref, dinv_ref, out_ref):
  dv = dinv_ref[...][:, 0:1]
  out_ref[...] = jnp.concatenate(
      [a20_ref[...] + y20_ref[...], a21_ref[...] + y21_ref[...]],
      axis=1) * dv


def kernel(x, edge_index, W1, W2):
  n, f_in = x.shape
  e = edge_index.shape[1]
  h = W1.shape[1]
  cdim = W2.shape[1]
  cpad = 128
  dh1 = h // 2
  dh2 = cpad // 2
  assert e % CH == 0
  nch = e // CH
  src2 = edge_index[0].reshape(nch, CH)
  dst2 = edge_index[1].reshape(nch, CH)
  srcs = jnp.stack([src2, src2 + n])
  W2p = jnp.zeros((h, cpad), jnp.float32).at[:, :cdim].set(W2)
  W2s = jnp.stack([W2p[:, :dh2], W2p[:, dh2:]])
  np_pad = -(-n // (8 * NS)) * (8 * NS)

  blk = 2000
  assert n % blk == 0
  nb = n // blk
  row = lambda width: pl.BlockSpec((blk, width), lambda c, i: (i, 0))
  plane = lambda width: pl.BlockSpec((1, blk, width), lambda c, i: (c, i, 0))
  wcol = lambda r, width: pl.BlockSpec((1, r, width), lambda c, i: (c, 0, 0))

  degp = _make_deg_kernel(np_pad, e)(dst2)

  rowf = lambda width: pl.BlockSpec((blk, width), lambda i: (i, 0))
  dinv, y1 = pl.pallas_call(
      _tc_layer1,
      grid=(nb,),
      in_specs=[rowf(L), rowf(L), rowf(f_in),
                pl.BlockSpec((f_in, h), lambda i: (0, 0))],
      out_specs=[rowf(L), rowf(h)],
      out_shape=[
          jax.ShapeDtypeStruct((n, L), jnp.float32),
          jax.ShapeDtypeStruct((n, h), jnp.float32),
      ],
  )(degp[0], degp[1], x, W1)

  aggp1 = _make_agg_kernel(np_pad, e, h, False)(y1, src2, dst2)

  y2 = pl.pallas_call(
      _tc_layer2,
      grid=(2, nb),
      in_specs=[row(h), row(h), row(h), row(L), wcol(h, dh2)],
      out_specs=plane(dh2),
      out_shape=jax.ShapeDtypeStruct((2, n, dh2), jnp.float32),
  )(aggp1[0], aggp1[1], y1, dinv, W2s)

  aggp2 = _make_agg_kernel(np_pad, e, dh2, True)(y2.reshape(2 * n, dh2),
                                                 srcs, dst2)

  rowf = lambda width: pl.BlockSpec((blk, width), lambda i: (i, 0))
  out = pl.pallas_call(
      _tc_final,
      grid=(nb,),
      in_specs=[rowf(dh2), rowf(dh2), rowf(dh2), rowf(dh2), rowf(L)],
      out_specs=rowf(cpad),
      out_shape=jax.ShapeDtypeStruct((n, cpad), jnp.float32),
  )(aggp2[0], aggp2[1], y2[0], y2[1], dinv)

  return out[:, :cdim]


# speedup vs baseline: 1.3826x; 1.2471x over previous
import functools

import jax
import jax.numpy as jnp
from jax import lax
from jax.experimental import pallas as pl
from jax.experimental.pallas import tpu as pltpu
from jax.experimental.pallas import tpu_sc as plsc

NC = 2
NS = 16
NW = NC * NS
L = 16
CH = 50


def _zero_rows(buf, nrows, ncols):
  z = jnp.zeros((L,), jnp.float32)

  def body(i, _):
    for jj in range(ncols // L):
      buf[i, pl.ds(jj * L, L)] = z
    return 0

  lax.fori_loop(0, nrows, body, 0)


def _fill_ones(buf, nrows, ncols):
  o = jnp.ones((L,), jnp.float32)

  def body(i, _):
    for jj in range(ncols // L):
      buf[i, pl.ds(jj * L, L)] = o
    return 0

  lax.fori_loop(0, nrows, body, 0)


def _make_deg_kernel(n, e):
  ring = 8
  assert e % (CH * NW) == 0 and n % (8 * NS) == 0
  nch = e // CH
  ncw = nch // NW
  assert ncw >= ring
  npt = n // NS
  zr = min(npt, CH)
  mesh = plsc.VectorSubcoreMesh(core_axis_name="c", subcore_axis_name="s")

  @functools.partial(
      pl.kernel,
      out_type=jax.ShapeDtypeStruct((NC, n, L), jnp.float32),
      mesh=mesh,
      compiler_params=pltpu.CompilerParams(use_tc_tiling_on_sc=False),
      scratch_types=[
          pltpu.VMEM_SHARED((n, L), jnp.float32),
          pltpu.VMEM((CH, L), jnp.float32),
          pltpu.VMEM((ncw, CH), jnp.int32),
          pltpu.SemaphoreType.DMA((ring,)),
      ],
  )
  def deg_kernel(dst2_hbm, degp_hbm, acc, buf, didx_all, ssem):
    c = lax.axis_index("c")
    s = lax.axis_index("s")
    w = c * NS + s
    row0 = s * npt

    pltpu.sync_copy(dst2_hbm.at[pl.ds(w * ncw, ncw)], didx_all)

    _zero_rows(buf, zr, L)
    off = 0
    while off < npt:
      step = min(zr, npt - off)
      pltpu.sync_copy(buf.at[pl.ds(0, step)],
                      acc.at[pl.ds(row0 + off, step)])
      off += step
    plsc.subcore_barrier()

    _fill_ones(buf, CH, L)
    for jj in range(ring):
      pltpu.async_copy(buf, acc.at[didx_all.at[jj]], ssem.at[jj], add=True)

    def body(j, _):
      rb = j % ring
      pltpu.make_async_copy(buf, acc.at[didx_all.at[j - ring]],
                            ssem.at[rb]).wait()
      pltpu.async_copy(buf, acc.at[didx_all.at[j]], ssem.at[rb], add=True)
      return 0

    lax.fori_loop(ring, ncw, body, 0)
    for k in range(ring):
      pltpu.make_async_copy(buf, acc.at[didx_all.at[ncw - ring + k]],
                            ssem.at[(ncw - ring + k) % ring]).wait()
    plsc.subcore_barrier()
    pltpu.sync_copy(acc.at[pl.ds(row0, npt)],
                    degp_hbm.at[c, pl.ds(row0, npt)])

  return deg_kernel


def _make_agg_kernel(n, e, d):
  assert e % (CH * NW) == 0 and n % (8 * NS) == 0 and d % L == 0
  nch = e // CH
  ncw = nch // NW
  npt = n // NS
  zr = min(npt, CH)
  ring = 8 if d <= 64 else 4
  assert ncw >= ring
  mesh = plsc.VectorSubcoreMesh(core_axis_name="c", subcore_axis_name="s")

  @functools.partial(
      pl.kernel,
      out_type=jax.ShapeDtypeStruct((NC, n, d), jnp.float32),
      mesh=mesh,
      compiler_params=pltpu.CompilerParams(use_tc_tiling_on_sc=False),
      scratch_types=[
          pltpu.VMEM_SHARED((n, d), jnp.float32),
          pltpu.VMEM((ring, CH, d), jnp.float32),
          pltpu.VMEM((ncw, CH), jnp.int32),
          pltpu.VMEM((ncw, CH), jnp.int32),
          pltpu.SemaphoreType.DMA((ring,)),
      ],
  )
  def agg_kernel(y_hbm, src2_hbm, dst2_hbm, aggp_hbm,
                 acc, rows_v, sidx_all, didx_all, gsem):
    c = lax.axis_index("c")
    s = lax.axis_index("s")
    w = c * NS + s
    row0 = s * npt

    pltpu.sync_copy(src2_hbm.at[pl.ds(w * ncw, ncw)], sidx_all)
    pltpu.sync_copy(dst2_hbm.at[pl.ds(w * ncw, ncw)], didx_all)

    zslot = rows_v.at[0]
    _zero_rows(zslot, zr, d)
    off = 0
    while off < npt:
      step = min(zr, npt - off)
      pltpu.sync_copy(zslot.at[pl.ds(0, step)],
                      acc.at[pl.ds(row0 + off, step)])
      off += step
    plsc.subcore_barrier()

    for jj in range(ring):
      pltpu.async_copy(y_hbm.at[sidx_all.at[jj]], rows_v.at[jj],
                       gsem.at[jj])

    def body(j, _):
      rb = j % ring
      pltpu.make_async_copy(y_hbm.at[sidx_all.at[j]], rows_v.at[rb],
                            gsem.at[rb]).wait()
      pltpu.sync_copy(rows_v.at[rb], acc.at[didx_all.at[j]], add=True)
      pltpu.async_copy(y_hbm.at[sidx_all.at[j + ring]], rows_v.at[rb],
                       gsem.at[rb])
      return 0

    lax.fori_loop(0, ncw - ring, body, 0)

    def tail(j, _):
      rb = j % ring
      pltpu.make_async_copy(y_hbm.at[sidx_all.at[j]], rows_v.at[rb],
                            gsem.at[rb]).wait()
      pltpu.sync_copy(rows_v.at[rb], acc.at[didx_all.at[j]], add=True)
      return 0

    lax.fori_loop(ncw - ring, ncw, tail, 0)
    plsc.subcore_barrier()
    pltpu.sync_copy(acc.at[pl.ds(row0, npt)],
                    aggp_hbm.at[c, pl.ds(row0, npt)])

  return agg_kernel


def _tc_layer1(degp0_ref, degp1_ref, x_ref, w1_ref, dinv_ref, y1_ref):
  deg = degp0_ref[...] + degp1_ref[...] + 1.0
  dinv = lax.rsqrt(deg)
  dinv_ref[...] = dinv
  xw = jnp.dot(x_ref[...], w1_ref[...], preferred_element_type=jnp.float32)
  y1_ref[...] = xw * dinv[:, 0:1]


def _tc_layer2(aggp0_ref, aggp1_ref, y1_ref, dinv_ref, w2_ref, y2_ref):
  dv = dinv_ref[...][:, 0:1]
  h = jnp.tanh((aggp0_ref[...] + aggp1_ref[...] + y1_ref[...]) * dv)
  y2_ref[...] = jnp.dot(h, w2_ref[...],
                        preferred_element_type=jnp.float32) * dv


def _tc_final(aggp0_ref, aggp1_ref, y2_ref, dinv_ref, out_ref):
  dv = dinv_ref[...][:, 0:1]
  out_ref[...] = (aggp0_ref[...] + aggp1_ref[...] + y2_ref[...]) * dv


def kernel(x, edge_index, W1, W2):
  n, f_in = x.shape
  e = edge_index.shape[1]
  h = W1.shape[1]
  cdim = W2.shape[1]
  cpad = 128
  assert e % CH == 0
  src2 = edge_index[0].reshape(e // CH, CH)
  dst2 = edge_index[1].reshape(e // CH, CH)
  W2p = jnp.zeros((h, cpad), jnp.float32).at[:, :cdim].set(W2)
  np_pad = -(-n // (8 * NS)) * (8 * NS)

  blk = 2000
  assert n % blk == 0
  grid = (n // blk,)
  row_spec = lambda width: pl.BlockSpec((blk, width), lambda i: (i, 0))
  full_spec = lambda r, ccol: pl.BlockSpec((r, ccol), lambda i: (0, 0))

  degp = _make_deg_kernel(np_pad, e)(dst2)

  dinv, y1 = pl.pallas_call(
      _tc_layer1,
      grid=grid,
      in_specs=[row_spec(L), row_spec(L), row_spec(f_in), full_spec(f_in, h)],
      out_specs=[row_spec(L), row_spec(h)],
      out_shape=[
          jax.ShapeDtypeStruct((n, L), jnp.float32),
          jax.ShapeDtypeStruct((n, h), jnp.float32),
      ],
  )(degp[0], degp[1], x, W1)

  aggp1 = _make_agg_kernel(np_pad, e, h)(y1, src2, dst2)

  y2 = pl.pallas_call(
      _tc_layer2,
      grid=grid,
      in_specs=[row_spec(h), row_spec(h), row_spec(h), row_spec(L),
                full_spec(h, cpad)],
      out_specs=row_spec(cpad),
      out_shape=jax.ShapeDtypeStruct((n, cpad), jnp.float32),
  )(aggp1[0], aggp1[1], y1, dinv, W2p)

  aggp2 = _make_agg_kernel(np_pad, e, cpad)(y2, src2, dst2)

  out = pl.pallas_call(
      _tc_final,
      grid=grid,
      in_specs=[row_spec(cpad), row_spec(cpad), row_spec(cpad), row_spec(L)],
      out_specs=row_spec(cpad),
      out_shape=jax.ShapeDtypeStruct((n, cpad), jnp.float32),
  )(aggp2[0], aggp2[1], y2, dinv)

  return out[:, :cdim]
